# Initial kernel scaffold; baseline (speedup 1.0000x reference)
import jax
import jax.numpy as jnp
from jax.experimental import pallas as pl


def kernel(F1, F2, fr, W_fr, W_last, b_last, W_att1, W_att2):
    raise NotImplementedError("write your pallas kernel here")


# speedup vs baseline: 11.7436x; 11.7436x over previous
import jax
import jax.numpy as jnp
from jax.experimental import pallas as pl
from jax.experimental.pallas import tpu as pltpu

B, C, H, W = 8, 96, 224, 224
K = 3
CB = 16
NC = C // CB


def _mean_body(f1_ref, f2_ref, m1_ref, m2_ref):
    m1_ref[...] = jnp.mean(f1_ref[...], axis=(2, 3)).reshape(1, 1, 1, CB)
    m2_ref[...] = jnp.mean(f2_ref[...], axis=(2, 3)).reshape(1, 1, 1, CB)


def _channel_means(F1, F2):
    out_sd = jax.ShapeDtypeStruct((B, NC, 1, CB), jnp.float32)
    m1, m2 = pl.pallas_call(
        _mean_body,
        grid=(B, NC),
        in_specs=[
            pl.BlockSpec((1, CB, H, W), lambda b, c: (b, c, 0, 0)),
            pl.BlockSpec((1, CB, H, W), lambda b, c: (b, c, 0, 0)),
        ],
        out_specs=[
            pl.BlockSpec((1, 1, 1, CB), lambda b, c: (b, c, 0, 0)),
            pl.BlockSpec((1, 1, 1, CB), lambda b, c: (b, c, 0, 0)),
        ],
        out_shape=[out_sd, out_sd],
        compiler_params=pltpu.CompilerParams(
            dimension_semantics=(pltpu.PARALLEL, pltpu.PARALLEL)),
    )(F1, F2)
    return m1.reshape(B, C), m2.reshape(B, C)


def _gating_body(fr_ref, x1_ref, x2_ref, wfr_ref,
                 ti1_ref, ti2_ref, w1_ref, w2_ref):
    fr_r = fr_ref[...].astype(jnp.bfloat16).astype(jnp.float32)
    wfr_r = wfr_ref[...].astype(jnp.bfloat16).astype(jnp.float32)
    mfr = jnp.mean(fr_r, axis=(2, 3))
    frp = (mfr[:, :, None] * wfr_r[None, :, :]).sum(axis=1)
    iota = jax.lax.broadcasted_iota(jnp.int32, (B, C), 1)

    def top3(dist, ti_ref, w_ref):
        d = dist
        vals, idxs = [], []
        for _ in range(K):
            v = jnp.max(d, axis=1, keepdims=True)
            hit = d == v
            idx = jnp.min(jnp.where(hit, iota, C), axis=1, keepdims=True)
            vals.append(v)
            idxs.append(idx)
            d = jnp.where(iota == idx, -jnp.inf, d)
        tv = jnp.concatenate(vals, axis=1)
        ti = jnp.concatenate(idxs, axis=1)
        e = jnp.exp(tv - tv[:, :1])
        w_ref[...] = e / jnp.sum(e, axis=1, keepdims=True)
        ti_ref[...] = ti.astype(jnp.int32)

    top3(-jnp.abs(frp - x1_ref[...]), ti1_ref, w1_ref)
    top3(-jnp.abs(frp - x2_ref[...]), ti2_ref, w2_ref)


def _gating(fr, x1, x2, wfr_t):
    return pl.pallas_call(
        _gating_body,
        out_shape=[
            jax.ShapeDtypeStruct((B, K), jnp.int32),
            jax.ShapeDtypeStruct((B, K), jnp.int32),
            jax.ShapeDtypeStruct((B, K), jnp.float32),
            jax.ShapeDtypeStruct((B, K), jnp.float32),
        ],
    )(fr, x1, x2, wfr_t)


def _fuse_body(ti1_ref, ti2_ref,
               f1_hbm, f2_hbm, w1_ref, w2_ref,
               wl_ref, bl_ref, wa1_ref, wa2_ref,
               out_ref,
               cat_ref, catp_ref, fusedp_ref, h_ref, sems):
    b = pl.program_id(0)
    copies = []
    for k in range(K):
        c1 = ti1_ref[b, k]
        cp = pltpu.make_async_copy(f1_hbm.at[b, c1], cat_ref.at[k], sems.at[k])
        cp.start()
        copies.append(cp)
    for k in range(K):
        c2 = ti2_ref[b, k]
        cp = pltpu.make_async_copy(f2_hbm.at[b, c2], cat_ref.at[K + k],
                                   sems.at[K + k])
        cp.start()
        copies.append(cp)

    catp_ref[...] = jnp.zeros_like(catp_ref)
    fusedp_ref[...] = jnp.zeros_like(fusedp_ref)

    for cp in copies:
        cp.wait()

    for k in range(K):
        catp_ref[k, 1:H + 1, 1:W + 1] = cat_ref[k] * w1_ref[b, k]
    for k in range(K):
        catp_ref[K + k, 1:H + 1, 1:W + 1] = cat_ref[K + k] * w2_ref[b, k]

    for o in range(3):
        acc = jnp.full((H, W), bl_ref[o], dtype=jnp.float32)
        for i in range(6):
            for dy in range(3):
                for dx in range(3):
                    wv = wl_ref[(o * 6 + i) * 9 + dy * 3 + dx]
                    acc = acc + catp_ref[i, dy:dy + H, dx:dx + W] * wv
        fusedp_ref[o, 1:H + 1, 1:W + 1] = acc

    for o in range(8):
        acc = jnp.zeros((H, W), dtype=jnp.float32)
        for i in range(3):
            for dy in range(3):
                for dx in range(3):
                    wv = wa1_ref[(o * 3 + i) * 9 + dy * 3 + dx]
                    acc = acc + fusedp_ref[i, dy:dy + H, dx:dx + W] * wv
        h_ref[o] = jnp.maximum(acc, 0.0)

    for o in range(3):
        acc = jnp.zeros((H, W), dtype=jnp.float32)
        for i in range(8):
            acc = acc + h_ref[i] * wa2_ref[o * 8 + i]
        out_ref[0, o] = jax.nn.sigmoid(acc)


def _fuse(F1, F2, ti1, ti2, w1, w2, wl, bl, wa1, wa2):
    grid_spec = pltpu.PrefetchScalarGridSpec(
        num_scalar_prefetch=2,
        grid=(B,),
        in_specs=[
            pl.BlockSpec(memory_space=pl.ANY),
            pl.BlockSpec(memory_space=pl.ANY),
            pl.BlockSpec(memory_space=pltpu.SMEM),
            pl.BlockSpec(memory_space=pltpu.SMEM),
            pl.BlockSpec(memory_space=pltpu.SMEM),
            pl.BlockSpec(memory_space=pltpu.SMEM),
            pl.BlockSpec(memory_space=pltpu.SMEM),
            pl.BlockSpec(memory_space=pltpu.SMEM),
        ],
        out_specs=pl.BlockSpec((1, 3, H, W), lambda b, t1, t2: (b, 0, 0, 0)),
        scratch_shapes=[
            pltpu.VMEM((6, H, W), jnp.float32),
            pltpu.VMEM((6, H + 2, W + 2), jnp.float32),
            pltpu.VMEM((3, H + 2, W + 2), jnp.float32),
            pltpu.VMEM((8, H, W), jnp.float32),
            pltpu.SemaphoreType.DMA((6,)),
        ],
    )
    return pl.pallas_call(
        _fuse_body,
        grid_spec=grid_spec,
        out_shape=jax.ShapeDtypeStruct((B, 3, H, W), jnp.float32),
    )(ti1, ti2, F1, F2, w1, w2, wl, bl, wa1, wa2)


def kernel(F1, F2, fr, W_fr, W_last, b_last, W_att1, W_att2):
    x1, x2 = _channel_means(F1, F2)
    wfr_t = W_fr.reshape(C, 3).T
    ti1, ti2, w1, w2 = _gating(fr, x1, x2, wfr_t)
    wl = W_last.reshape(-1)
    wa1 = W_att1.reshape(-1)
    wa2 = W_att2.reshape(-1)
    return _fuse(F1, F2, ti1, ti2, w1, w2, wl, b_last, wa1, wa2)
